# baseline (speedup 1.0000x reference)
import functools

import jax
import jax.numpy as jnp
from jax import lax
from jax.experimental import pallas as pl
from jax.experimental.pallas import tpu as pltpu
from jax.experimental.pallas import tpu_sc as plsc

D = 128
NC = 2
NS = 16
CH = 64
NB = 5
IB = 8
DL = 2


def _agg_body(with_count, nchunks, rows_per_tile, n_pad,
              table_a, table_b, sa3, da3, sb3, db3, zrows_h, zcnt_h,
              ones_h, *rest):
    if with_count:
        out_a, out_b, cnt_a, cnt_b = rest[:4]
        rest = rest[4:]
    else:
        out_a, out_b = rest[:2]
        cnt_a = cnt_b = None
        rest = rest[2:]
    (s0_v, s1_v, d0_v, d1_v, ones_v, z1_v, acc_sh, cnt_sh) = rest[:8]
    src_v = (s0_v, s1_v)
    dst_v = (d0_v, d1_v)
    rows = rest[8:8 + NB]
    gsem = rest[8 + NB:8 + 2 * NB]
    ssem = rest[8 + 2 * NB:8 + 3 * NB]
    csem = rest[8 + 3 * NB:8 + 3 * NB + 2]
    c = lax.axis_index("c")
    s = lax.axis_index("s")

    pltpu.sync_copy(zrows_h, rows[0])
    pltpu.sync_copy(ones_h, ones_v)
    pltpu.sync_copy(zcnt_h, z1_v)

    row0 = pl.multiple_of(s * rows_per_tile, rows_per_tile)
    for j in range(rows_per_tile // CH):
        pltpu.sync_copy(rows[0], acc_sh.at[pl.ds(row0 + j * CH, CH)])
        pltpu.sync_copy(z1_v, cnt_sh.at[pl.ds(row0 + j * CH, CH)])
    plsc.subcore_barrier()

    def run_relation(table, src3, dst3):
        def load_idx(kb):
            sl = pl.ds(kb * IB, IB)
            pltpu.sync_copy(src3.at[s, sl], src_v[kb % 2])
            pltpu.sync_copy(dst3.at[s, sl], dst_v[kb % 2])

        def gather(k, b):
            idx = src_v[(k // IB) % 2].at[k % IB]
            return pltpu.async_copy(table.at[idx], rows[b], gsem[b])

        def scatter(k, b):
            idx = dst_v[(k // IB) % 2].at[k % IB]
            return pltpu.async_copy(rows[b], acc_sh.at[idx], ssem[b],
                                    add=True)

        def count(k):
            idx = dst_v[(k // IB) % 2].at[k % IB]
            return pltpu.async_copy(ones_v, cnt_sh.at[idx], csem[k % 2],
                                    add=True)

        gd = [None] * NB
        sd = [None] * NB
        cd = [None, None]
        unwaited = set()
        load_idx(0)
        for b in range(min(NB, nchunks)):
            gd[b] = gather(b, b)
        for k in range(nchunks):
            b = k % NB
            gd[b].wait()
            if k % IB == DL and (k // IB + 1) * IB < nchunks:
                load_idx(k // IB + 1)
            if with_count:
                if cd[k % 2] is not None:
                    cd[k % 2].wait()
                cd[k % 2] = count(k)
            sd[b] = scatter(k, b)
            unwaited.add(b)
            pk, nk = k - DL, k - DL + NB
            if pk >= 0 and nk < nchunks:
                pb = pk % NB
                sd[pb].wait()
                unwaited.discard(pb)
                gd[pb] = gather(nk, pb)
        for b in sorted(unwaited):
            sd[b].wait()
        for x in cd:
            if x is not None:
                x.wait()

    @pl.when(c == 0)
    def _():
        run_relation(table_a, sa3, da3)

    @pl.when(c == 1)
    def _():
        run_relation(table_b, sb3, db3)

    plsc.subcore_barrier()

    @pl.when(c == 0)
    def _():
        pltpu.sync_copy(acc_sh.at[pl.ds(row0, rows_per_tile)],
                        out_a.at[pl.ds(row0, rows_per_tile)])
        if with_count:
            pltpu.sync_copy(cnt_sh.at[pl.ds(row0, rows_per_tile)],
                            cnt_a.at[pl.ds(row0, rows_per_tile)])

    @pl.when(c == 1)
    def _():
        pltpu.sync_copy(acc_sh.at[pl.ds(row0, rows_per_tile)],
                        out_b.at[pl.ds(row0, rows_per_tile)])
        if with_count:
            pltpu.sync_copy(cnt_sh.at[pl.ds(row0, rows_per_tile)],
                            cnt_b.at[pl.ds(row0, rows_per_tile)])


@functools.lru_cache(maxsize=None)
def _make_agg(n_pad, nchunks, with_count):
    rows_per_tile = n_pad // NS
    assert rows_per_tile % CH == 0
    mesh = plsc.VectorSubcoreMesh(core_axis_name="c", subcore_axis_name="s",
                                  num_cores=NC, num_subcores=NS)
    out_type = [jax.ShapeDtypeStruct((n_pad, D), jnp.float32),
                jax.ShapeDtypeStruct((n_pad, D), jnp.float32)]
    if with_count:
        out_type += [jax.ShapeDtypeStruct((n_pad,), jnp.float32),
                     jax.ShapeDtypeStruct((n_pad,), jnp.float32)]
    scratch = [
        pltpu.VMEM((IB, CH), jnp.int32),
        pltpu.VMEM((IB, CH), jnp.int32),
        pltpu.VMEM((IB, CH), jnp.int32),
        pltpu.VMEM((IB, CH), jnp.int32),
        pltpu.VMEM((CH,), jnp.float32),
        pltpu.VMEM((CH,), jnp.float32),
        pltpu.VMEM_SHARED((n_pad, D), jnp.float32),
        pltpu.VMEM_SHARED((n_pad,), jnp.float32),
    ]
    scratch += [pltpu.VMEM((CH, D), jnp.float32) for _ in range(NB)]
    scratch += [pltpu.SemaphoreType.DMA for _ in range(2 * NB + 2)]
    body = functools.partial(_agg_body, with_count, nchunks, rows_per_tile,
                             n_pad)
    return pl.kernel(body, out_type=tuple(out_type), mesh=mesh,
                     scratch_types=tuple(scratch))


def _dense_body(leaky, x_ref, p_ref, c_ref, w_ref, b_ref, o_ref):
    inv = 1.0 / jnp.maximum(c_ref[:], 1.0)
    hn = p_ref[:] * inv[:, None]
    xx = jnp.concatenate([x_ref[:], hn], axis=1)
    h = jnp.dot(xx, w_ref[:], preferred_element_type=jnp.float32)
    h = h + b_ref[:]
    if leaky:
        h = jnp.where(h >= 0, h, 0.01 * h)
    o_ref[...] = h


def _dense(x, p, cnt, w_cat, b, leaky, block_rows=1024):
    n = x.shape[0]
    assert n % block_rows == 0
    return pl.pallas_call(
        functools.partial(_dense_body, leaky),
        grid=(n // block_rows,),
        in_specs=[
            pl.BlockSpec((block_rows, D), lambda i: (i, 0)),
            pl.BlockSpec((block_rows, D), lambda i: (i, 0)),
            pl.BlockSpec((block_rows,), lambda i: (i,)),
            pl.BlockSpec((2 * D, D), lambda i: (0, 0)),
            pl.BlockSpec((D,), lambda i: (0,)),
        ],
        out_specs=pl.BlockSpec((block_rows, D), lambda i: (i, 0)),
        out_shape=jax.ShapeDtypeStruct((n, D), jnp.float32),
    )(x, p, cnt, w_cat, b)


def _round_up(a, m):
    return (a + m - 1) // m * m


def kernel(edge_uv, edge_vu, emb_user, emb_item,
           W1_uv_self, W1_uv_neigh, b1_uv, W1_vu_self, W1_vu_neigh, b1_vu,
           W2_uv_self, W2_uv_neigh, b2_uv, W2_vu_self, W2_vu_neigh, b2_vu):
    n_user, n_item = emb_user.shape[0], emb_item.shape[0]
    e = edge_uv.shape[1]
    n_pad = _round_up(max(n_user, n_item), NS * CH)
    e_per_tile = _round_up(-(-e // NS), CH * IB)
    nchunks = e_per_tile // CH
    e_pad = NS * e_per_tile

    def _prep_idx(v):
        return jnp.pad(v.astype(jnp.int32), (0, e_pad - e),
                       constant_values=n_pad - 1).reshape(NS, nchunks, CH)

    si3, du3 = _prep_idx(edge_vu[0]), _prep_idx(edge_vu[1])
    su3, di3 = _prep_idx(edge_uv[0]), _prep_idx(edge_uv[1])

    xu = jnp.zeros((n_pad, D), jnp.float32).at[:n_user].set(emb_user)
    xi = jnp.zeros((n_pad, D), jnp.float32).at[:n_item].set(emb_item)

    agg_c = _make_agg(n_pad, nchunks, True)
    agg = _make_agg(n_pad, nchunks, False)

    zrows = jnp.zeros((CH, D), jnp.float32)
    zcnt = jnp.zeros((CH,), jnp.float32)
    ones = jnp.ones((CH,), jnp.float32)

    p1u, p1i, cu, ci = agg_c(xi, xu, si3, du3, su3, di3,
                             zrows, zcnt, ones)

    w1_uv = jnp.concatenate([W1_uv_self, W1_uv_neigh], axis=0)
    w1_vu = jnp.concatenate([W1_vu_self, W1_vu_neigh], axis=0)
    w2_uv = jnp.concatenate([W2_uv_self, W2_uv_neigh], axis=0)
    w2_vu = jnp.concatenate([W2_vu_self, W2_vu_neigh], axis=0)

    h1_item = _dense(xi, p1i, ci, w1_uv, b1_uv, leaky=True)
    h1_user = _dense(xu, p1u, cu, w1_vu, b1_vu, leaky=True)

    p2u, p2i = agg(h1_item, h1_user, si3, du3, su3, di3,
                   zrows, zcnt, ones)
    h2_item = _dense(h1_item, p2i, ci, w2_uv, b2_uv, leaky=False)
    h2_user = _dense(h1_user, p2u, cu, w2_vu, b2_vu, leaky=False)
    return (h2_user[:n_user], h2_item[:n_item])


# speedup vs baseline: 1.0090x; 1.0090x over previous
import functools

import jax
import jax.numpy as jnp
from jax import lax
from jax.experimental import pallas as pl
from jax.experimental.pallas import tpu as pltpu
from jax.experimental.pallas import tpu_sc as plsc

D = 128
NC = 2
NS = 16
CH = 64
NB = 5
IB = 8
DL = 1


def _agg_body(with_count, nchunks, rows_per_tile, n_pad,
              table_a, table_b, sa3, da3, sb3, db3, zrows_h, zcnt_h,
              ones_h, *rest):
    if with_count:
        out_a, out_b, cnt_a, cnt_b = rest[:4]
        rest = rest[4:]
    else:
        out_a, out_b = rest[:2]
        cnt_a = cnt_b = None
        rest = rest[2:]
    (s0_v, s1_v, d0_v, d1_v, ones_v, z1_v, acc_sh, cnt_sh) = rest[:8]
    src_v = (s0_v, s1_v)
    dst_v = (d0_v, d1_v)
    rows = rest[8:8 + NB]
    gsem = rest[8 + NB:8 + 2 * NB]
    ssem = rest[8 + 2 * NB:8 + 3 * NB]
    csem = rest[8 + 3 * NB:8 + 3 * NB + 2]
    c = lax.axis_index("c")
    s = lax.axis_index("s")

    pltpu.sync_copy(zrows_h, rows[0])
    pltpu.sync_copy(ones_h, ones_v)
    pltpu.sync_copy(zcnt_h, z1_v)

    row0 = pl.multiple_of(s * rows_per_tile, rows_per_tile)
    for j in range(rows_per_tile // CH):
        pltpu.sync_copy(rows[0], acc_sh.at[pl.ds(row0 + j * CH, CH)])
        pltpu.sync_copy(z1_v, cnt_sh.at[pl.ds(row0 + j * CH, CH)])
    plsc.subcore_barrier()

    def run_relation(table, src3, dst3):
        def load_idx(kb):
            sl = pl.ds(kb * IB, IB)
            pltpu.sync_copy(src3.at[s, sl], src_v[kb % 2])
            pltpu.sync_copy(dst3.at[s, sl], dst_v[kb % 2])

        def gather(k, b):
            idx = src_v[(k // IB) % 2].at[k % IB]
            return pltpu.async_copy(table.at[idx], rows[b], gsem[b])

        def scatter(k, b):
            idx = dst_v[(k // IB) % 2].at[k % IB]
            return pltpu.async_copy(rows[b], acc_sh.at[idx], ssem[b],
                                    add=True)

        def count(k):
            idx = dst_v[(k // IB) % 2].at[k % IB]
            return pltpu.async_copy(ones_v, cnt_sh.at[idx], csem[k % 2],
                                    add=True)

        gd = [None] * NB
        sd = [None] * NB
        cd = [None, None]
        unwaited = set()
        load_idx(0)
        for b in range(min(NB, nchunks)):
            gd[b] = gather(b, b)
        for k in range(nchunks):
            b = k % NB
            gd[b].wait()
            if k % IB == DL and (k // IB + 1) * IB < nchunks:
                load_idx(k // IB + 1)
            if with_count:
                idx = dst_v[(k // IB) % 2].at[k % IB]
                pltpu.sync_copy(ones_v, cnt_sh.at[idx], add=True)
            sd[b] = scatter(k, b)
            unwaited.add(b)
            pk, nk = k - DL, k - DL + NB
            if pk >= 0 and nk < nchunks:
                pb = pk % NB
                sd[pb].wait()
                unwaited.discard(pb)
                gd[pb] = gather(nk, pb)
        for b in sorted(unwaited):
            sd[b].wait()
        for x in cd:
            if x is not None:
                x.wait()

    @pl.when(c == 0)
    def _():
        run_relation(table_a, sa3, da3)

    @pl.when(c == 1)
    def _():
        run_relation(table_b, sb3, db3)

    plsc.subcore_barrier()

    @pl.when(c == 0)
    def _():
        pltpu.sync_copy(acc_sh.at[pl.ds(row0, rows_per_tile)],
                        out_a.at[pl.ds(row0, rows_per_tile)])
        if with_count:
            pltpu.sync_copy(cnt_sh.at[pl.ds(row0, rows_per_tile)],
                            cnt_a.at[pl.ds(row0, rows_per_tile)])

    @pl.when(c == 1)
    def _():
        pltpu.sync_copy(acc_sh.at[pl.ds(row0, rows_per_tile)],
                        out_b.at[pl.ds(row0, rows_per_tile)])
        if with_count:
            pltpu.sync_copy(cnt_sh.at[pl.ds(row0, rows_per_tile)],
                            cnt_b.at[pl.ds(row0, rows_per_tile)])


@functools.lru_cache(maxsize=None)
def _make_agg(n_pad, nchunks, with_count):
    rows_per_tile = n_pad // NS
    assert rows_per_tile % CH == 0
    mesh = plsc.VectorSubcoreMesh(core_axis_name="c", subcore_axis_name="s",
                                  num_cores=NC, num_subcores=NS)
    out_type = [jax.ShapeDtypeStruct((n_pad, D), jnp.float32),
                jax.ShapeDtypeStruct((n_pad, D), jnp.float32)]
    if with_count:
        out_type += [jax.ShapeDtypeStruct((n_pad,), jnp.float32),
                     jax.ShapeDtypeStruct((n_pad,), jnp.float32)]
    scratch = [
        pltpu.VMEM((IB, CH), jnp.int32),
        pltpu.VMEM((IB, CH), jnp.int32),
        pltpu.VMEM((IB, CH), jnp.int32),
        pltpu.VMEM((IB, CH), jnp.int32),
        pltpu.VMEM((CH,), jnp.float32),
        pltpu.VMEM((CH,), jnp.float32),
        pltpu.VMEM_SHARED((n_pad, D), jnp.float32),
        pltpu.VMEM_SHARED((n_pad,), jnp.float32),
    ]
    scratch += [pltpu.VMEM((CH, D), jnp.float32) for _ in range(NB)]
    scratch += [pltpu.SemaphoreType.DMA for _ in range(2 * NB + 2)]
    body = functools.partial(_agg_body, with_count, nchunks, rows_per_tile,
                             n_pad)
    return pl.kernel(body, out_type=tuple(out_type), mesh=mesh,
                     scratch_types=tuple(scratch))


def _dense_body(leaky, x_ref, p_ref, c_ref, w_ref, b_ref, o_ref):
    inv = 1.0 / jnp.maximum(c_ref[:], 1.0)
    hn = p_ref[:] * inv[:, None]
    xx = jnp.concatenate([x_ref[:], hn], axis=1)
    h = jnp.dot(xx, w_ref[:], preferred_element_type=jnp.float32)
    h = h + b_ref[:]
    if leaky:
        h = jnp.where(h >= 0, h, 0.01 * h)
    o_ref[...] = h


def _dense(x, p, cnt, w_cat, b, leaky, block_rows=1024):
    n = x.shape[0]
    assert n % block_rows == 0
    return pl.pallas_call(
        functools.partial(_dense_body, leaky),
        grid=(n // block_rows,),
        in_specs=[
            pl.BlockSpec((block_rows, D), lambda i: (i, 0)),
            pl.BlockSpec((block_rows, D), lambda i: (i, 0)),
            pl.BlockSpec((block_rows,), lambda i: (i,)),
            pl.BlockSpec((2 * D, D), lambda i: (0, 0)),
            pl.BlockSpec((D,), lambda i: (0,)),
        ],
        out_specs=pl.BlockSpec((block_rows, D), lambda i: (i, 0)),
        out_shape=jax.ShapeDtypeStruct((n, D), jnp.float32),
    )(x, p, cnt, w_cat, b)


def _round_up(a, m):
    return (a + m - 1) // m * m


def kernel(edge_uv, edge_vu, emb_user, emb_item,
           W1_uv_self, W1_uv_neigh, b1_uv, W1_vu_self, W1_vu_neigh, b1_vu,
           W2_uv_self, W2_uv_neigh, b2_uv, W2_vu_self, W2_vu_neigh, b2_vu):
    n_user, n_item = emb_user.shape[0], emb_item.shape[0]
    e = edge_uv.shape[1]
    n_pad = _round_up(max(n_user, n_item), NS * CH)
    e_per_tile = _round_up(-(-e // NS), CH * IB)
    nchunks = e_per_tile // CH
    e_pad = NS * e_per_tile

    def _prep_idx(v):
        return jnp.pad(v.astype(jnp.int32), (0, e_pad - e),
                       constant_values=n_pad - 1).reshape(NS, nchunks, CH)

    si3, du3 = _prep_idx(edge_vu[0]), _prep_idx(edge_vu[1])
    su3, di3 = _prep_idx(edge_uv[0]), _prep_idx(edge_uv[1])

    xu = jnp.zeros((n_pad, D), jnp.float32).at[:n_user].set(emb_user)
    xi = jnp.zeros((n_pad, D), jnp.float32).at[:n_item].set(emb_item)

    agg_c = _make_agg(n_pad, nchunks, True)
    agg = _make_agg(n_pad, nchunks, False)

    zrows = jnp.zeros((CH, D), jnp.float32)
    zcnt = jnp.zeros((CH,), jnp.float32)
    ones = jnp.ones((CH,), jnp.float32)

    p1u, p1i, cu, ci = agg_c(xi, xu, si3, du3, su3, di3,
                             zrows, zcnt, ones)

    w1_uv = jnp.concatenate([W1_uv_self, W1_uv_neigh], axis=0)
    w1_vu = jnp.concatenate([W1_vu_self, W1_vu_neigh], axis=0)
    w2_uv = jnp.concatenate([W2_uv_self, W2_uv_neigh], axis=0)
    w2_vu = jnp.concatenate([W2_vu_self, W2_vu_neigh], axis=0)

    h1_item = _dense(xi, p1i, ci, w1_uv, b1_uv, leaky=True)
    h1_user = _dense(xu, p1u, cu, w1_vu, b1_vu, leaky=True)

    p2u, p2i = agg(h1_item, h1_user, si3, du3, su3, di3,
                   zrows, zcnt, ones)
    h2_item = _dense(h1_item, p2i, ci, w2_uv, b2_uv, leaky=False)
    h2_user = _dense(h1_user, p2u, cu, w2_vu, b2_vu, leaky=False)
    return (h2_user[:n_user], h2_item[:n_item])


# speedup vs baseline: 1.2229x; 1.2119x over previous
import functools

import jax
import jax.numpy as jnp
from jax import lax
from jax.experimental import pallas as pl
from jax.experimental.pallas import tpu as pltpu
from jax.experimental.pallas import tpu_sc as plsc

D = 128
NC = 2
NS = 16
CH = 64
NB = 5
IB = 8
DL = 1


def _agg_body(with_count, nchunks, rows_per_tile, n_pad,
              table_a, table_b, sa3, da3, sb3, db3, zrows_h, zcnt_h,
              ones_h, *rest):
    if with_count:
        out_a, out_b, cnt_a, cnt_b = rest[:4]
        rest = rest[4:]
    else:
        out_a, out_b = rest[:2]
        cnt_a = cnt_b = None
        rest = rest[2:]
    (s0_v, s1_v, d0_v, d1_v, ones_v, z1_v, acc_sh, cnt_sh) = rest[:8]
    src_v = (s0_v, s1_v)
    dst_v = (d0_v, d1_v)
    rows = rest[8:8 + NB]
    gsem = rest[8 + NB:8 + 2 * NB]
    ssem = rest[8 + 2 * NB:8 + 3 * NB]
    csem = rest[8 + 3 * NB:8 + 3 * NB + 2]
    c = lax.axis_index("c")
    s = lax.axis_index("s")

    pltpu.sync_copy(zrows_h, rows[0])
    pltpu.sync_copy(ones_h, ones_v)
    pltpu.sync_copy(zcnt_h, z1_v)

    row0 = pl.multiple_of(s * rows_per_tile, rows_per_tile)
    for j in range(rows_per_tile // CH):
        pltpu.sync_copy(rows[0], acc_sh.at[pl.ds(row0 + j * CH, CH)])
        pltpu.sync_copy(z1_v, cnt_sh.at[pl.ds(row0 + j * CH, CH)])
    plsc.subcore_barrier()

    def run_relation(table, src3, dst3):
        def load_idx(kb):
            sl = pl.ds(kb * IB, IB)
            pltpu.sync_copy(src3.at[s, sl], src_v[kb % 2])
            pltpu.sync_copy(dst3.at[s, sl], dst_v[kb % 2])

        def gather(k, b):
            idx = src_v[(k // IB) % 2].at[k % IB]
            return pltpu.async_copy(table.at[idx], rows[b], gsem[b])

        def scatter(k, b):
            idx = dst_v[(k // IB) % 2].at[k % IB]
            return pltpu.async_copy(rows[b], acc_sh.at[idx], ssem[b],
                                    add=True)

        def count(k):
            idx = dst_v[(k // IB) % 2].at[k % IB]
            return pltpu.async_copy(ones_v, cnt_sh.at[idx], csem[k % 2],
                                    add=True)

        gd = [None] * NB
        sd = [None] * NB
        cd = [None, None]
        unwaited = set()
        load_idx(0)
        for b in range(min(NB, nchunks)):
            gd[b] = gather(b, b)
        for k in range(nchunks):
            b = k % NB
            gd[b].wait()
            if k % IB == DL and (k // IB + 1) * IB < nchunks:
                load_idx(k // IB + 1)
            if with_count:
                idx = dst_v[(k // IB) % 2].at[k % IB]
                pltpu.sync_copy(ones_v, cnt_sh.at[idx], add=True)
            sd[b] = scatter(k, b)
            unwaited.add(b)
            pk, nk = k - DL, k - DL + NB
            if pk >= 0 and nk < nchunks:
                pb = pk % NB
                sd[pb].wait()
                unwaited.discard(pb)
                gd[pb] = gather(nk, pb)
        for b in sorted(unwaited):
            sd[b].wait()
        for x in cd:
            if x is not None:
                x.wait()

    @pl.when(c == 0)
    def _():
        run_relation(table_a, sa3, da3)

    @pl.when(c == 1)
    def _():
        run_relation(table_b, sb3, db3)

    plsc.subcore_barrier()

    @pl.when(c == 0)
    def _():
        pltpu.sync_copy(acc_sh.at[pl.ds(row0, rows_per_tile)],
                        out_a.at[pl.ds(row0, rows_per_tile)])
        if with_count:
            pltpu.sync_copy(cnt_sh.at[pl.ds(row0, rows_per_tile)],
                            cnt_a.at[pl.ds(row0, rows_per_tile)])

    @pl.when(c == 1)
    def _():
        pltpu.sync_copy(acc_sh.at[pl.ds(row0, rows_per_tile)],
                        out_b.at[pl.ds(row0, rows_per_tile)])
        if with_count:
            pltpu.sync_copy(cnt_sh.at[pl.ds(row0, rows_per_tile)],
                            cnt_b.at[pl.ds(row0, rows_per_tile)])


@functools.lru_cache(maxsize=None)
def _make_agg(n_pad, nchunks, with_count):
    rows_per_tile = n_pad // NS
    assert rows_per_tile % CH == 0
    mesh = plsc.VectorSubcoreMesh(core_axis_name="c", subcore_axis_name="s",
                                  num_cores=NC, num_subcores=NS)
    out_type = [jax.ShapeDtypeStruct((n_pad, D), jnp.float32),
                jax.ShapeDtypeStruct((n_pad, D), jnp.float32)]
    if with_count:
        out_type += [jax.ShapeDtypeStruct((n_pad,), jnp.float32),
                     jax.ShapeDtypeStruct((n_pad,), jnp.float32)]
    scratch = [
        pltpu.VMEM((IB, CH), jnp.int32),
        pltpu.VMEM((IB, CH), jnp.int32),
        pltpu.VMEM((IB, CH), jnp.int32),
        pltpu.VMEM((IB, CH), jnp.int32),
        pltpu.VMEM((CH,), jnp.float32),
        pltpu.VMEM((CH,), jnp.float32),
        pltpu.VMEM_SHARED((n_pad, D), jnp.float32),
        pltpu.VMEM_SHARED((n_pad,), jnp.float32),
    ]
    scratch += [pltpu.VMEM((CH, D), jnp.float32) for _ in range(NB)]
    scratch += [pltpu.SemaphoreType.DMA for _ in range(2 * NB + 2)]
    body = functools.partial(_agg_body, with_count, nchunks, rows_per_tile,
                             n_pad)
    return pl.kernel(body, out_type=tuple(out_type), mesh=mesh,
                     scratch_types=tuple(scratch))


def _dense_body(leaky, x_ref, p_ref, c_ref, w_ref, b_ref, o_ref):
    inv = 1.0 / jnp.maximum(c_ref[:], 1.0)
    hn = p_ref[:] * inv[:, None]
    xx = jnp.concatenate([x_ref[:], hn], axis=1)
    h = jnp.dot(xx, w_ref[:], preferred_element_type=jnp.float32)
    h = h + b_ref[:]
    if leaky:
        h = jnp.where(h >= 0, h, 0.01 * h)
    o_ref[...] = h


def _dense(x, p, cnt, w_cat, b, leaky, block_rows=1024):
    n = x.shape[0]
    assert n % block_rows == 0
    return pl.pallas_call(
        functools.partial(_dense_body, leaky),
        grid=(n // block_rows,),
        in_specs=[
            pl.BlockSpec((block_rows, D), lambda i: (i, 0)),
            pl.BlockSpec((block_rows, D), lambda i: (i, 0)),
            pl.BlockSpec((block_rows,), lambda i: (i,)),
            pl.BlockSpec((2 * D, D), lambda i: (0, 0)),
            pl.BlockSpec((D,), lambda i: (0,)),
        ],
        out_specs=pl.BlockSpec((block_rows, D), lambda i: (i, 0)),
        out_shape=jax.ShapeDtypeStruct((n, D), jnp.float32),
    )(x, p, cnt, w_cat, b)


def _round_up(a, m):
    return (a + m - 1) // m * m


def kernel(edge_uv, edge_vu, emb_user, emb_item,
           W1_uv_self, W1_uv_neigh, b1_uv, W1_vu_self, W1_vu_neigh, b1_vu,
           W2_uv_self, W2_uv_neigh, b2_uv, W2_vu_self, W2_vu_neigh, b2_vu):
    n_user, n_item = emb_user.shape[0], emb_item.shape[0]
    e = edge_uv.shape[1]
    n_pad = _round_up(max(n_user, n_item), NS * CH)
    e_per_tile = _round_up(-(-e // NS), CH * IB)
    nchunks = e_per_tile // CH
    e_pad = NS * e_per_tile

    def _prep_idx(v):
        return jnp.pad(v.astype(jnp.int32), (0, e_pad - e),
                       constant_values=n_pad - 1).reshape(NS, nchunks, CH)

    si3, du3 = _prep_idx(edge_vu[0]), _prep_idx(edge_vu[1])
    su3, di3 = _prep_idx(edge_uv[0]), _prep_idx(edge_uv[1])

    xu = jnp.zeros((n_pad, D), jnp.float32).at[:n_user].set(emb_user)
    xi = jnp.zeros((n_pad, D), jnp.float32).at[:n_item].set(emb_item)

    agg_c = _make_agg(n_pad, nchunks, True)
    agg = _make_agg(n_pad, nchunks, False)

    zrows = jnp.zeros((CH, D), jnp.float32)
    zcnt = jnp.zeros((CH,), jnp.float32)
    ones = jnp.ones((CH,), jnp.float32)

    p1i, p1u, ci, cu = agg_c(xu, xi, su3, di3, si3, du3,
                             zrows, zcnt, ones)

    w1_uv = jnp.concatenate([W1_uv_self, W1_uv_neigh], axis=0)
    w1_vu = jnp.concatenate([W1_vu_self, W1_vu_neigh], axis=0)
    w2_uv = jnp.concatenate([W2_uv_self, W2_uv_neigh], axis=0)
    w2_vu = jnp.concatenate([W2_vu_self, W2_vu_neigh], axis=0)

    h1_item = _dense(xi, p1i, ci, w1_uv, b1_uv, leaky=True)
    h1_user = _dense(xu, p1u, cu, w1_vu, b1_vu, leaky=True)

    p2i, p2u = agg(h1_user, h1_item, su3, di3, si3, du3,
                   zrows, zcnt, ones)
    h2_item = _dense(h1_item, p2i, ci, w2_uv, b2_uv, leaky=False)
    h2_user = _dense(h1_user, p2u, cu, w2_vu, b2_vu, leaky=False)
    return (h2_user[:n_user], h2_item[:n_item])


# speedup vs baseline: 1.2230x; 1.0001x over previous
import functools

import jax
import jax.numpy as jnp
from jax import lax
from jax.experimental import pallas as pl
from jax.experimental.pallas import tpu as pltpu
from jax.experimental.pallas import tpu_sc as plsc

D = 128
NC = 2
NS = 16
CH = 64
NB = 5
IB = 8
DL = 1


def _agg_body(with_count, nchunks, rows_per_tile, n_pad,
              table_a, table_b, sa3, da3, sb3, db3, zrows_h, zcnt_h,
              ones_h, *rest):
    if with_count:
        out_a, out_b, cnt_a, cnt_b = rest[:4]
        rest = rest[4:]
    else:
        out_a, out_b = rest[:2]
        cnt_a = cnt_b = None
        rest = rest[2:]
    (s0_v, s1_v, d0_v, d1_v, ones_v, z1_v, acc_sh, cnt_sh) = rest[:8]
    src_v = (s0_v, s1_v)
    dst_v = (d0_v, d1_v)
    rows = rest[8:8 + NB]
    gsem = rest[8 + NB:8 + 2 * NB]
    ssem = rest[8 + 2 * NB:8 + 3 * NB]
    c = lax.axis_index("c")
    s = lax.axis_index("s")

    pltpu.sync_copy(zrows_h, rows[0])
    pltpu.sync_copy(ones_h, ones_v)
    pltpu.sync_copy(zcnt_h, z1_v)

    row0 = pl.multiple_of(s * rows_per_tile, rows_per_tile)
    for j in range(rows_per_tile // CH):
        pltpu.sync_copy(rows[0], acc_sh.at[pl.ds(row0 + j * CH, CH)])
        pltpu.sync_copy(z1_v, cnt_sh.at[pl.ds(row0 + j * CH, CH)])
    plsc.subcore_barrier()

    def run_relation(table, src3, dst3):
        def load_idx(kb):
            sl = pl.ds(kb * IB, IB)
            pltpu.sync_copy(src3.at[s, sl], src_v[kb % 2])
            pltpu.sync_copy(dst3.at[s, sl], dst_v[kb % 2])

        def gather(k, b):
            idx = src_v[(k // IB) % 2].at[k % IB]
            return pltpu.async_copy(table.at[idx], rows[b], gsem[b])

        def scatter(k, b):
            idx = dst_v[(k // IB) % 2].at[k % IB]
            return pltpu.async_copy(rows[b], acc_sh.at[idx], ssem[b],
                                    add=True)

        gd = [None] * NB
        sd = [None] * NB
        unwaited = set()
        load_idx(0)
        for b in range(min(NB, nchunks)):
            gd[b] = gather(b, b)
        for k in range(nchunks):
            b = k % NB
            gd[b].wait()
            if k % IB == DL and (k // IB + 1) * IB < nchunks:
                load_idx(k // IB + 1)
            if with_count:
                idx = dst_v[(k // IB) % 2].at[k % IB]
                pltpu.sync_copy(ones_v, cnt_sh.at[idx], add=True)
            sd[b] = scatter(k, b)
            unwaited.add(b)
            pk, nk = k - DL, k - DL + NB
            if pk >= 0 and nk < nchunks:
                pb = pk % NB
                sd[pb].wait()
                unwaited.discard(pb)
                gd[pb] = gather(nk, pb)
        for b in sorted(unwaited):
            sd[b].wait()

    @pl.when(c == 0)
    def _():
        run_relation(table_a, sa3, da3)

    @pl.when(c == 1)
    def _():
        run_relation(table_b, sb3, db3)

    plsc.subcore_barrier()

    @pl.when(c == 0)
    def _():
        pltpu.sync_copy(acc_sh.at[pl.ds(row0, rows_per_tile)],
                        out_a.at[pl.ds(row0, rows_per_tile)])
        if with_count:
            pltpu.sync_copy(cnt_sh.at[pl.ds(row0, rows_per_tile)],
                            cnt_a.at[pl.ds(row0, rows_per_tile)])

    @pl.when(c == 1)
    def _():
        pltpu.sync_copy(acc_sh.at[pl.ds(row0, rows_per_tile)],
                        out_b.at[pl.ds(row0, rows_per_tile)])
        if with_count:
            pltpu.sync_copy(cnt_sh.at[pl.ds(row0, rows_per_tile)],
                            cnt_b.at[pl.ds(row0, rows_per_tile)])


@functools.lru_cache(maxsize=None)
def _make_agg(n_pad, nchunks, with_count):
    rows_per_tile = n_pad // NS
    assert rows_per_tile % CH == 0
    mesh = plsc.VectorSubcoreMesh(core_axis_name="c", subcore_axis_name="s",
                                  num_cores=NC, num_subcores=NS)
    out_type = [jax.ShapeDtypeStruct((n_pad, D), jnp.float32),
                jax.ShapeDtypeStruct((n_pad, D), jnp.float32)]
    if with_count:
        out_type += [jax.ShapeDtypeStruct((n_pad,), jnp.float32),
                     jax.ShapeDtypeStruct((n_pad,), jnp.float32)]
    scratch = [
        pltpu.VMEM((IB, CH), jnp.int32),
        pltpu.VMEM((IB, CH), jnp.int32),
        pltpu.VMEM((IB, CH), jnp.int32),
        pltpu.VMEM((IB, CH), jnp.int32),
        pltpu.VMEM((CH,), jnp.float32),
        pltpu.VMEM((CH,), jnp.float32),
        pltpu.VMEM_SHARED((n_pad, D), jnp.float32),
        pltpu.VMEM_SHARED((n_pad,), jnp.float32),
    ]
    scratch += [pltpu.VMEM((CH, D), jnp.float32) for _ in range(NB)]
    scratch += [pltpu.SemaphoreType.DMA for _ in range(2 * NB)]
    body = functools.partial(_agg_body, with_count, nchunks, rows_per_tile,
                             n_pad)
    return pl.kernel(body, out_type=tuple(out_type), mesh=mesh,
                     scratch_types=tuple(scratch))


def _dense_body(leaky, x_ref, p_ref, c_ref, w_ref, b_ref, o_ref):
    inv = 1.0 / jnp.maximum(c_ref[:], 1.0)
    hn = p_ref[:] * inv[:, None]
    xx = jnp.concatenate([x_ref[:], hn], axis=1)
    h = jnp.dot(xx, w_ref[:], preferred_element_type=jnp.float32)
    h = h + b_ref[:]
    if leaky:
        h = jnp.where(h >= 0, h, 0.01 * h)
    o_ref[...] = h


def _dense(x, p, cnt, w_cat, b, leaky, block_rows=1024):
    n = x.shape[0]
    assert n % block_rows == 0
    return pl.pallas_call(
        functools.partial(_dense_body, leaky),
        grid=(n // block_rows,),
        in_specs=[
            pl.BlockSpec((block_rows, D), lambda i: (i, 0)),
            pl.BlockSpec((block_rows, D), lambda i: (i, 0)),
            pl.BlockSpec((block_rows,), lambda i: (i,)),
            pl.BlockSpec((2 * D, D), lambda i: (0, 0)),
            pl.BlockSpec((D,), lambda i: (0,)),
        ],
        out_specs=pl.BlockSpec((block_rows, D), lambda i: (i, 0)),
        out_shape=jax.ShapeDtypeStruct((n, D), jnp.float32),
    )(x, p, cnt, w_cat, b)


def _round_up(a, m):
    return (a + m - 1) // m * m


def kernel(edge_uv, edge_vu, emb_user, emb_item,
           W1_uv_self, W1_uv_neigh, b1_uv, W1_vu_self, W1_vu_neigh, b1_vu,
           W2_uv_self, W2_uv_neigh, b2_uv, W2_vu_self, W2_vu_neigh, b2_vu):
    n_user, n_item = emb_user.shape[0], emb_item.shape[0]
    e = edge_uv.shape[1]
    n_pad = _round_up(max(n_user, n_item), NS * CH)
    e_per_tile = _round_up(-(-e // NS), CH * IB)
    nchunks = e_per_tile // CH
    e_pad = NS * e_per_tile

    def _prep_idx(v):
        return jnp.pad(v.astype(jnp.int32), (0, e_pad - e),
                       constant_values=n_pad - 1).reshape(NS, nchunks, CH)

    si3, du3 = _prep_idx(edge_vu[0]), _prep_idx(edge_vu[1])
    su3, di3 = _prep_idx(edge_uv[0]), _prep_idx(edge_uv[1])

    xu = jnp.zeros((n_pad, D), jnp.float32).at[:n_user].set(emb_user)
    xi = jnp.zeros((n_pad, D), jnp.float32).at[:n_item].set(emb_item)

    agg_c = _make_agg(n_pad, nchunks, True)
    agg = _make_agg(n_pad, nchunks, False)

    zrows = jnp.zeros((CH, D), jnp.float32)
    zcnt = jnp.zeros((CH,), jnp.float32)
    ones = jnp.ones((CH,), jnp.float32)

    p1i, p1u, ci, cu = agg_c(xu, xi, su3, di3, si3, du3,
                             zrows, zcnt, ones)

    w1_uv = jnp.concatenate([W1_uv_self, W1_uv_neigh], axis=0)
    w1_vu = jnp.concatenate([W1_vu_self, W1_vu_neigh], axis=0)
    w2_uv = jnp.concatenate([W2_uv_self, W2_uv_neigh], axis=0)
    w2_vu = jnp.concatenate([W2_vu_self, W2_vu_neigh], axis=0)

    h1_item = _dense(xi, p1i, ci, w1_uv, b1_uv, leaky=True)
    h1_user = _dense(xu, p1u, cu, w1_vu, b1_vu, leaky=True)

    p2i, p2u = agg(h1_user, h1_item, su3, di3, si3, du3,
                   zrows, zcnt, ones)
    h2_item = _dense(h1_item, p2i, ci, w2_uv, b2_uv, leaky=False)
    h2_user = _dense(h1_user, p2u, cu, w2_vu, b2_vu, leaky=False)
    return (h2_user[:n_user], h2_item[:n_item])
